# Initial kernel scaffold; baseline (speedup 1.0000x reference)
import jax
import jax.numpy as jnp
from jax.experimental import pallas as pl


def kernel(x, w_gate, w1, w2):
    raise NotImplementedError("write your pallas kernel here")


# speedup vs baseline: 1.3855x; 1.3855x over previous
import functools

import jax
import jax.numpy as jnp
from jax.experimental import pallas as pl
from jax.experimental.pallas import tpu as pltpu

T = 2048
D_MODEL = 768
D_FF = 3072
NUM_EXPERTS = 16
TOP_K = 2
LOSS_COEF = 0.01


def _routing_kernel(x_ref, wg_ref, combine_ref, aux_ref):
    x = x_ref[...]
    wg = wg_ref[...]
    logits = jnp.dot(x, wg, preferred_element_type=jnp.float32)
    m = jnp.max(logits, axis=-1, keepdims=True)
    unnorm = jnp.exp(logits - m)
    gates = unnorm / jnp.sum(unnorm, axis=-1, keepdims=True)

    lane = jax.lax.broadcasted_iota(jnp.int32, (T, NUM_EXPERTS), 1)
    big = jnp.int32(NUM_EXPERTS)

    m0 = jnp.max(gates, axis=-1, keepdims=True)
    i0 = jnp.min(jnp.where(gates == m0, lane, big), axis=-1, keepdims=True)
    masked = jnp.where(lane == i0, -jnp.inf, gates)
    m1 = jnp.max(masked, axis=-1, keepdims=True)
    i1 = jnp.min(jnp.where(masked == m1, lane, big), axis=-1, keepdims=True)

    denom = m0 + m1 + 1e-9
    v0 = m0 / denom
    v1 = m1 / denom
    combine = (jnp.where(lane == i0, v0, 0.0)
               + jnp.where(lane == i1, v1, 0.0))
    combine_ref[...] = combine

    importance = jnp.sum(gates, axis=0, keepdims=True)
    load = jnp.sum((combine > 0.0).astype(jnp.float32), axis=0, keepdims=True)

    def cv_sq(v):
        mean = jnp.mean(v)
        var = jnp.mean((v - mean) ** 2)
        return var / (mean * mean + 1e-10)

    aux = LOSS_COEF * (cv_sq(importance) + cv_sq(load))
    aux_ref[...] = jnp.broadcast_to(aux, (1, 1))


FF_CHUNK = 1024
N_FF_CHUNKS = D_FF // FF_CHUNK


def _ffn_kernel(x_ref, w1_ref, w2_ref, combine_ref, out_ref):
    e = pl.program_id(0)
    ci = pl.program_id(1)
    x = x_ref[...]
    c = combine_ref[...]
    lane = jax.lax.broadcasted_iota(jnp.int32, (T, NUM_EXPERTS), 1)
    col = jnp.sum(jnp.where(lane == e, c, 0.0), axis=-1, keepdims=True)

    @pl.when((e == 0) & (ci == 0))
    def _():
        out_ref[...] = jnp.zeros_like(out_ref)

    h = jnp.maximum(jnp.dot(x, w1_ref[0], preferred_element_type=jnp.float32), 0.0)
    y = jnp.dot(h, w2_ref[0], preferred_element_type=jnp.float32)
    out_ref[...] += col * y


@jax.jit
def kernel(x, w_gate, w1, w2):
    combine, aux = pl.pallas_call(
        _routing_kernel,
        out_shape=[
            jax.ShapeDtypeStruct((T, NUM_EXPERTS), jnp.float32),
            jax.ShapeDtypeStruct((1, 1), jnp.float32),
        ],
    )(x, w_gate)

    out = pl.pallas_call(
        _ffn_kernel,
        grid=(NUM_EXPERTS, N_FF_CHUNKS),
        in_specs=[
            pl.BlockSpec((T, D_MODEL), lambda e, c: (0, 0)),
            pl.BlockSpec((1, D_MODEL, FF_CHUNK), lambda e, c: (e, 0, c)),
            pl.BlockSpec((1, FF_CHUNK, D_MODEL), lambda e, c: (e, c, 0)),
            pl.BlockSpec((T, NUM_EXPERTS), lambda e, c: (0, 0)),
        ],
        out_specs=pl.BlockSpec((T, D_MODEL), lambda e, c: (0, 0)),
        out_shape=jax.ShapeDtypeStruct((T, D_MODEL), jnp.float32),
        compiler_params=pltpu.CompilerParams(
            dimension_semantics=("arbitrary", "arbitrary"),
        ),
    )(x, w1, w2, combine)

    return out, aux[0, 0]


# speedup vs baseline: 1.5532x; 1.1211x over previous
import functools

import jax
import jax.numpy as jnp
from jax import lax
from jax.experimental import pallas as pl
from jax.experimental.pallas import tpu as pltpu
from jax.experimental.pallas import tpu_sc as plsc

T = 2048
D_MODEL = 768
D_FF = 3072
NUM_EXPERTS = 16
LOSS_COEF = 0.01

B = 128
NB = (2 * T) // B + NUM_EXPERTS
NP = NB * B
PC = 512
NC = NP // PC

SC_CORES = 2
SC_SUBCORES = 16
NW = SC_CORES * SC_SUBCORES


def _routing_kernel(x_ref, wg_ref, pos0_ref, pos1_ref, sid_ref, sgw_ref,
                    bexp_ref, bval_ref, aux_ref):
    x = x_ref[...]
    logits = jnp.dot(x, wg_ref[...], preferred_element_type=jnp.float32)
    m = jnp.max(logits, axis=-1, keepdims=True)
    unnorm = jnp.exp(logits - m)
    gates = unnorm / jnp.sum(unnorm, axis=-1, keepdims=True)

    lane = lax.broadcasted_iota(jnp.int32, (T, NUM_EXPERTS), 1)
    m0 = jnp.max(gates, axis=-1, keepdims=True)
    i0 = jnp.min(jnp.where(gates == m0, lane, NUM_EXPERTS), axis=-1,
                 keepdims=True)
    masked = jnp.where(lane == i0, -jnp.inf, gates)
    m1 = jnp.max(masked, axis=-1, keepdims=True)
    i1 = jnp.min(jnp.where(masked == m1, lane, NUM_EXPERTS), axis=-1,
                 keepdims=True)
    denom = m0 + m1 + 1e-9
    v0 = m0 / denom
    v1 = m1 / denom

    ind = ((lane == i0) | (lane == i1)).astype(jnp.float32)
    counts = jnp.sum(ind, axis=0, keepdims=True)
    padded = jnp.floor((counts + (B - 1)) / B) * B

    erow = lax.broadcasted_iota(jnp.int32, (NUM_EXPERTS, NUM_EXPERTS), 0)
    ecol = lax.broadcasted_iota(jnp.int32, (NUM_EXPERTS, NUM_EXPERTS), 1)
    strict = (erow < ecol).astype(jnp.float32)
    offs = jnp.dot(padded, strict, preferred_element_type=jnp.float32)

    RC = 512
    prior_parts = []
    for rc in range(T // RC):
        ri = lax.broadcasted_iota(jnp.int32, (RC, T), 0) + rc * RC
        ci = lax.broadcasted_iota(jnp.int32, (RC, T), 1)
        tri = (ci < ri).astype(jnp.float32)
        prior_parts.append(
            jnp.dot(tri, ind, preferred_element_type=jnp.float32))
    prior = jnp.concatenate(prior_parts, axis=0)

    posm = offs + prior
    pos0 = jnp.sum(jnp.where(lane == i0, posm, 0.0), axis=-1, keepdims=True)
    pos1 = jnp.sum(jnp.where(lane == i1, posm, 0.0), axis=-1, keepdims=True)
    pos0_ref[...] = pos0.astype(jnp.int32)
    pos1_ref[...] = pos1.astype(jnp.int32)

    bstart = lax.broadcasted_iota(jnp.int32, (NB, NUM_EXPERTS), 0).astype(jnp.float32) * B
    onehot = ((offs <= bstart) & (bstart < offs + padded)).astype(jnp.float32)
    lane_f = lax.broadcasted_iota(jnp.int32, (NB, NUM_EXPERTS), 1).astype(jnp.float32)
    be = jnp.sum(onehot * lane_f, axis=-1, keepdims=True)
    valid = jnp.sum(onehot, axis=-1, keepdims=True)
    bexp_ref[...] = jnp.where(valid > 0, be, NUM_EXPERTS - 1).astype(jnp.int32)
    bval_ref[...] = (valid > 0).astype(jnp.int32)

    tcol = lax.broadcasted_iota(jnp.int32, (T, 1), 0).astype(jnp.float32)

    def chunk_body(c, _):
        prow = (c * PC).astype(jnp.float32) + lax.broadcasted_iota(
            jnp.int32, (1, PC), 1).astype(jnp.float32)
        m0c = pos0 == prow
        m1c = pos1 == prow
        sel = (m0c | m1c).astype(jnp.float32)
        ids = jnp.sum(tcol * sel, axis=0, keepdims=True)
        gws = jnp.sum(jnp.where(m0c, v0, 0.0) + jnp.where(m1c, v1, 0.0),
                      axis=0, keepdims=True)
        sid_ref[pl.ds(c, 1), :] = ids.astype(jnp.int32)
        sgw_ref[pl.ds(c, 1), :] = gws
        return 0

    lax.fori_loop(0, NC, chunk_body, 0)

    importance = jnp.sum(gates, axis=0, keepdims=True)
    combine_v = (jnp.where(lane == i0, v0, 0.0)
                 + jnp.where(lane == i1, v1, 0.0))
    load = jnp.sum((combine_v > 0.0).astype(jnp.float32), axis=0,
                   keepdims=True)

    def cv_sq(v):
        mean = jnp.mean(v)
        var = jnp.mean((v - mean) ** 2)
        return var / (mean * mean + 1e-10)

    aux = LOSS_COEF * (cv_sq(importance) + cv_sq(load))
    aux_ref[...] = jnp.broadcast_to(aux, (1, 1))


def _routing_call(x, w_gate):
    return pl.pallas_call(
        _routing_kernel,
        out_shape=[
            jax.ShapeDtypeStruct((T, 1), jnp.int32),
            jax.ShapeDtypeStruct((T, 1), jnp.int32),
            jax.ShapeDtypeStruct((NC, PC), jnp.int32),
            jax.ShapeDtypeStruct((NC, PC), jnp.float32),
            jax.ShapeDtypeStruct((NB, 1), jnp.int32),
            jax.ShapeDtypeStruct((NB, 1), jnp.int32),
            jax.ShapeDtypeStruct((1, 1), jnp.float32),
        ],
    )(x, w_gate)


def _ffn_kernel(bexp_ref, bval_ref, xs_ref, w1_ref, w2_ref, gw_ref, y_ref):
    b = pl.program_id(0)

    @pl.when(bval_ref[b] != 0)
    def _():
        h = jnp.maximum(
            jnp.dot(xs_ref[...], w1_ref[0], preferred_element_type=jnp.float32),
            0.0)
        y = jnp.dot(h, w2_ref[0], preferred_element_type=jnp.float32)
        blane = lax.broadcasted_iota(jnp.int32, (B, NB), 1)
        gw = jnp.sum(jnp.where(blane == b, gw_ref[...], 0.0), axis=-1,
                     keepdims=True)
        y_ref[...] = y * gw


def _ffn_call(bexp, bval, xs, w1, w2, gw_col):
    grid_spec = pltpu.PrefetchScalarGridSpec(
        num_scalar_prefetch=2,
        grid=(NB,),
        in_specs=[
            pl.BlockSpec((B, D_MODEL), lambda b, be, bv: (b, 0)),
            pl.BlockSpec((1, D_MODEL, D_FF), lambda b, be, bv: (be[b], 0, 0)),
            pl.BlockSpec((1, D_FF, D_MODEL), lambda b, be, bv: (be[b], 0, 0)),
            pl.BlockSpec((B, NB), lambda b, be, bv: (0, 0)),
        ],
        out_specs=pl.BlockSpec((B, D_MODEL), lambda b, be, bv: (b, 0)),
    )
    return pl.pallas_call(
        _ffn_kernel,
        grid_spec=grid_spec,
        out_shape=jax.ShapeDtypeStruct((NP, D_MODEL), jnp.float32),
        compiler_params=pltpu.CompilerParams(
            dimension_semantics=("arbitrary",),
        ),
    )(bexp, bval, xs, w1, w2, gw_col)


def _sc_gather(x, ids_flat):
    rows_per_w = NP // NW
    CH = 64
    n_ch = rows_per_w // CH
    mesh = plsc.VectorSubcoreMesh(core_axis_name="c", subcore_axis_name="s")

    @functools.partial(
        pl.kernel, mesh=mesh,
        out_type=jax.ShapeDtypeStruct((NP, D_MODEL), jnp.float32),
        scratch_types=[
            pltpu.VMEM((CH,), jnp.int32),
            pltpu.VMEM((CH, D_MODEL), jnp.float32),
            pltpu.SemaphoreType.DMA,
        ],
    )
    def k(x_hbm, ids_hbm, out_hbm, idx_v, rows_v, sem):
        wid = lax.axis_index("s") * SC_CORES + lax.axis_index("c")
        base = wid * rows_per_w
        for ci in range(n_ch):
            off = base + ci * CH
            pltpu.sync_copy(ids_hbm.at[pl.ds(off, CH)], idx_v)
            pltpu.async_copy(x_hbm.at[idx_v], rows_v, sem).wait()
            pltpu.sync_copy(rows_v, out_hbm.at[pl.ds(off, CH)])

    return k(x, ids_flat)


def _sc_combine(yw, pos0, pos1):
    tok_per_w = T // NW
    mesh = plsc.VectorSubcoreMesh(core_axis_name="c", subcore_axis_name="s")

    @functools.partial(
        pl.kernel, mesh=mesh,
        out_type=jax.ShapeDtypeStruct((T, D_MODEL), jnp.float32),
        scratch_types=[
            pltpu.VMEM((tok_per_w,), jnp.int32),
            pltpu.VMEM((tok_per_w,), jnp.int32),
            pltpu.VMEM((tok_per_w, D_MODEL), jnp.float32),
            pltpu.VMEM((tok_per_w, D_MODEL), jnp.float32),
            pltpu.SemaphoreType.DMA,
            pltpu.SemaphoreType.DMA,
        ],
    )
    def k(yw_hbm, p0_hbm, p1_hbm, out_hbm, i0v, i1v, r0, r1, sem0, sem1):
        wid = lax.axis_index("s") * SC_CORES + lax.axis_index("c")
        base = wid * tok_per_w
        pltpu.sync_copy(p0_hbm.at[pl.ds(base, tok_per_w)], i0v)
        pltpu.sync_copy(p1_hbm.at[pl.ds(base, tok_per_w)], i1v)
        cp0 = pltpu.async_copy(yw_hbm.at[i0v], r0, sem0)
        cp1 = pltpu.async_copy(yw_hbm.at[i1v], r1, sem1)
        cp0.wait()
        cp1.wait()

        def row_add(r, carry):
            for l in range(D_MODEL // 16):
                sl = pl.ds(l * 16, 16)
                r0[r, sl] = r0[r, sl] + r1[r, sl]
            return carry

        lax.fori_loop(0, tok_per_w, row_add, 0)
        pltpu.sync_copy(r0, out_hbm.at[pl.ds(base, tok_per_w)])

    return k(yw, pos0, pos1)


@jax.jit
def kernel(x, w_gate, w1, w2):
    pos0, pos1, sid, sgw, bexp, bval, aux = _routing_call(x, w_gate)
    ids_flat = sid.reshape(NP)
    gw_col = sgw.reshape(NB, B).T
    xs = _sc_gather(x, ids_flat)
    yw = _ffn_call(bexp.reshape(NB), bval.reshape(NB), xs, w1, w2, gw_col)
    out = _sc_combine(yw, pos0.reshape(T), pos1.reshape(T))
    return out, aux[0, 0]


# speedup vs baseline: 1.5601x; 1.0044x over previous
import functools

import jax
import jax.numpy as jnp
from jax import lax
from jax.experimental import pallas as pl
from jax.experimental.pallas import tpu as pltpu
from jax.experimental.pallas import tpu_sc as plsc

T = 2048
D_MODEL = 768
D_FF = 3072
NUM_EXPERTS = 16
LOSS_COEF = 0.01

B = 128
NB = (2 * T) // B + NUM_EXPERTS
NP = NB * B
PC = 512
NC = NP // PC

SC_CORES = 2
SC_SUBCORES = 16
NW = SC_CORES * SC_SUBCORES


def _routing_kernel(x_ref, wg_ref, pos0_ref, pos1_ref, sid_ref, sgw_ref,
                    bexp_ref, bval_ref, aux_ref):
    x = x_ref[...]
    logits = jnp.dot(x, wg_ref[...], preferred_element_type=jnp.float32)
    m = jnp.max(logits, axis=-1, keepdims=True)
    unnorm = jnp.exp(logits - m)
    gates = unnorm / jnp.sum(unnorm, axis=-1, keepdims=True)

    lane = lax.broadcasted_iota(jnp.int32, (T, NUM_EXPERTS), 1)
    m0 = jnp.max(gates, axis=-1, keepdims=True)
    i0 = jnp.min(jnp.where(gates == m0, lane, NUM_EXPERTS), axis=-1,
                 keepdims=True)
    masked = jnp.where(lane == i0, -jnp.inf, gates)
    m1 = jnp.max(masked, axis=-1, keepdims=True)
    i1 = jnp.min(jnp.where(masked == m1, lane, NUM_EXPERTS), axis=-1,
                 keepdims=True)
    denom = m0 + m1 + 1e-9
    v0 = m0 / denom
    v1 = m1 / denom

    ind = ((lane == i0) | (lane == i1)).astype(jnp.float32)
    counts = jnp.sum(ind, axis=0, keepdims=True)
    padded = jnp.floor((counts + (B - 1)) / B) * B

    erow = lax.broadcasted_iota(jnp.int32, (NUM_EXPERTS, NUM_EXPERTS), 0)
    ecol = lax.broadcasted_iota(jnp.int32, (NUM_EXPERTS, NUM_EXPERTS), 1)
    strict = (erow < ecol).astype(jnp.float32)
    offs = jnp.dot(padded, strict, preferred_element_type=jnp.float32)

    RC = 512
    prior_parts = []
    for rc in range(T // RC):
        ri = lax.broadcasted_iota(jnp.int32, (RC, T), 0) + rc * RC
        ci = lax.broadcasted_iota(jnp.int32, (RC, T), 1)
        tri = (ci < ri).astype(jnp.float32)
        prior_parts.append(
            jnp.dot(tri, ind, preferred_element_type=jnp.float32))
    prior = jnp.concatenate(prior_parts, axis=0)

    posm = offs + prior
    pos0 = jnp.sum(jnp.where(lane == i0, posm, 0.0), axis=-1, keepdims=True)
    pos1 = jnp.sum(jnp.where(lane == i1, posm, 0.0), axis=-1, keepdims=True)
    pos0_ref[...] = pos0.astype(jnp.int32)
    pos1_ref[...] = pos1.astype(jnp.int32)

    bstart = lax.broadcasted_iota(jnp.int32, (NB, NUM_EXPERTS), 0).astype(jnp.float32) * B
    onehot = ((offs <= bstart) & (bstart < offs + padded)).astype(jnp.float32)
    lane_f = lax.broadcasted_iota(jnp.int32, (NB, NUM_EXPERTS), 1).astype(jnp.float32)
    be = jnp.sum(onehot * lane_f, axis=-1, keepdims=True)
    valid = jnp.sum(onehot, axis=-1, keepdims=True)
    bexp_ref[...] = jnp.where(valid > 0, be, NUM_EXPERTS - 1).astype(jnp.int32)
    bval_ref[...] = (valid > 0).astype(jnp.int32)

    tcol = lax.broadcasted_iota(jnp.int32, (T, 1), 0).astype(jnp.float32)

    def chunk_body(c, _):
        prow = (c * PC).astype(jnp.float32) + lax.broadcasted_iota(
            jnp.int32, (1, PC), 1).astype(jnp.float32)
        m0c = pos0 == prow
        m1c = pos1 == prow
        sel = (m0c | m1c).astype(jnp.float32)
        ids = jnp.sum(tcol * sel, axis=0, keepdims=True)
        gws = jnp.sum(jnp.where(m0c, v0, 0.0) + jnp.where(m1c, v1, 0.0),
                      axis=0, keepdims=True)
        sid_ref[pl.ds(c, 1), :] = ids.astype(jnp.int32)
        sgw_ref[pl.ds(c, 1), :] = gws
        return 0

    lax.fori_loop(0, NC, chunk_body, 0)

    importance = jnp.sum(gates, axis=0, keepdims=True)
    combine_v = (jnp.where(lane == i0, v0, 0.0)
                 + jnp.where(lane == i1, v1, 0.0))
    load = jnp.sum((combine_v > 0.0).astype(jnp.float32), axis=0,
                   keepdims=True)

    def cv_sq(v):
        mean = jnp.mean(v)
        var = jnp.mean((v - mean) ** 2)
        return var / (mean * mean + 1e-10)

    aux = LOSS_COEF * (cv_sq(importance) + cv_sq(load))
    aux_ref[...] = jnp.broadcast_to(aux, (1, 1))


def _routing_call(x, w_gate):
    return pl.pallas_call(
        _routing_kernel,
        out_shape=[
            jax.ShapeDtypeStruct((T, 1), jnp.int32),
            jax.ShapeDtypeStruct((T, 1), jnp.int32),
            jax.ShapeDtypeStruct((NC, PC), jnp.int32),
            jax.ShapeDtypeStruct((NC, PC), jnp.float32),
            jax.ShapeDtypeStruct((NB, 1), jnp.int32),
            jax.ShapeDtypeStruct((NB, 1), jnp.int32),
            jax.ShapeDtypeStruct((1, 1), jnp.float32),
        ],
    )(x, w_gate)


def _ffn_kernel(bexp_ref, bval_ref, xs_ref, w1_ref, w2_ref, gw_ref, y_ref):
    b = pl.program_id(0)

    @pl.when(bval_ref[b] != 0)
    def _():
        xb = xs_ref[...].astype(jnp.bfloat16)
        h = jnp.maximum(
            jnp.dot(xb, w1_ref[0].astype(jnp.bfloat16),
                    preferred_element_type=jnp.float32),
            0.0).astype(jnp.bfloat16)
        y = jnp.dot(h, w2_ref[0].astype(jnp.bfloat16),
                    preferred_element_type=jnp.float32)
        blane = lax.broadcasted_iota(jnp.int32, (B, NB), 1)
        gw = jnp.sum(jnp.where(blane == b, gw_ref[...], 0.0), axis=-1,
                     keepdims=True)
        y_ref[...] = y * gw


def _ffn_call(bexp, bval, xs, w1, w2, gw_col):
    grid_spec = pltpu.PrefetchScalarGridSpec(
        num_scalar_prefetch=2,
        grid=(NB,),
        in_specs=[
            pl.BlockSpec((B, D_MODEL), lambda b, be, bv: (b, 0)),
            pl.BlockSpec((1, D_MODEL, D_FF), lambda b, be, bv: (be[b], 0, 0)),
            pl.BlockSpec((1, D_FF, D_MODEL), lambda b, be, bv: (be[b], 0, 0)),
            pl.BlockSpec((B, NB), lambda b, be, bv: (0, 0)),
        ],
        out_specs=pl.BlockSpec((B, D_MODEL), lambda b, be, bv: (b, 0)),
    )
    return pl.pallas_call(
        _ffn_kernel,
        grid_spec=grid_spec,
        out_shape=jax.ShapeDtypeStruct((NP, D_MODEL), jnp.float32),
        compiler_params=pltpu.CompilerParams(
            dimension_semantics=("arbitrary",),
        ),
    )(bexp, bval, xs, w1, w2, gw_col)


def _sc_gather(x, ids_flat):
    rows_per_w = NP // NW
    CH = 64
    n_ch = rows_per_w // CH
    mesh = plsc.VectorSubcoreMesh(core_axis_name="c", subcore_axis_name="s")

    @functools.partial(
        pl.kernel, mesh=mesh,
        out_type=jax.ShapeDtypeStruct((NP, D_MODEL), jnp.float32),
        scratch_types=[
            pltpu.VMEM((CH,), jnp.int32),
            pltpu.VMEM((CH, D_MODEL), jnp.float32),
            pltpu.SemaphoreType.DMA,
        ],
    )
    def k(x_hbm, ids_hbm, out_hbm, idx_v, rows_v, sem):
        wid = lax.axis_index("s") * SC_CORES + lax.axis_index("c")
        base = wid * rows_per_w
        for ci in range(n_ch):
            off = base + ci * CH
            pltpu.sync_copy(ids_hbm.at[pl.ds(off, CH)], idx_v)
            pltpu.async_copy(x_hbm.at[idx_v], rows_v, sem).wait()
            pltpu.sync_copy(rows_v, out_hbm.at[pl.ds(off, CH)])

    return k(x, ids_flat)


def _sc_combine(yw, pos0, pos1):
    tok_per_w = T // NW
    mesh = plsc.VectorSubcoreMesh(core_axis_name="c", subcore_axis_name="s")

    @functools.partial(
        pl.kernel, mesh=mesh,
        out_type=jax.ShapeDtypeStruct((T, D_MODEL), jnp.float32),
        scratch_types=[
            pltpu.VMEM((tok_per_w,), jnp.int32),
            pltpu.VMEM((tok_per_w,), jnp.int32),
            pltpu.VMEM((tok_per_w, D_MODEL), jnp.float32),
            pltpu.VMEM((tok_per_w, D_MODEL), jnp.float32),
            pltpu.SemaphoreType.DMA,
            pltpu.SemaphoreType.DMA,
        ],
    )
    def k(yw_hbm, p0_hbm, p1_hbm, out_hbm, i0v, i1v, r0, r1, sem0, sem1):
        wid = lax.axis_index("s") * SC_CORES + lax.axis_index("c")
        base = wid * tok_per_w
        pltpu.sync_copy(p0_hbm.at[pl.ds(base, tok_per_w)], i0v)
        pltpu.sync_copy(p1_hbm.at[pl.ds(base, tok_per_w)], i1v)
        cp0 = pltpu.async_copy(yw_hbm.at[i0v], r0, sem0)
        cp1 = pltpu.async_copy(yw_hbm.at[i1v], r1, sem1)
        cp0.wait()
        cp1.wait()

        def row_add(r, carry):
            for l in range(D_MODEL // 16):
                sl = pl.ds(l * 16, 16)
                r0[r, sl] = r0[r, sl] + r1[r, sl]
            return carry

        lax.fori_loop(0, tok_per_w, row_add, 0)
        pltpu.sync_copy(r0, out_hbm.at[pl.ds(base, tok_per_w)])

    return k(yw, pos0, pos1)


@jax.jit
def kernel(x, w_gate, w1, w2):
    pos0, pos1, sid, sgw, bexp, bval, aux = _routing_call(x, w_gate)
    ids_flat = sid.reshape(NP)
    gw_col = sgw.reshape(NB, B).T
    xs = _sc_gather(x, ids_flat)
    yw = _ffn_call(bexp.reshape(NB), bval.reshape(NB), xs, w1, w2, gw_col)
    out = _sc_combine(yw, pos0.reshape(T), pos1.reshape(T))
    return out, aux[0, 0]
